# baseline (device time: 109759 ns/iter reference)
import jax
import jax.numpy as jnp
from jax import lax
from jax.experimental import pallas as pl
from jax.experimental.pallas import tpu as pltpu

N_DEV = 16
N_TOK = 1024
D_MODEL = 256
N_EXP = 64
H = 512
CHUNK = N_TOK // N_DEV


def kernel(x, router_W, route_idx, expert_W):
    e_local = expert_W.shape[0]

    def body(x_ref, rw_ref, idx_ref, ew_ref, out_ref,
             acc, rs_buf, rs_send, rs_recv, ag_send, ag_recv):
        d = lax.axis_index("i")
        left = lax.rem(d + N_DEV - 1, N_DEV)
        right = lax.rem(d + 1, N_DEV)

        barrier_sem = pltpu.get_barrier_semaphore()
        for nbr in (left, right):
            pl.semaphore_signal(
                barrier_sem, inc=1,
                device_id=(nbr,), device_id_type=pl.DeviceIdType.MESH,
            )
        pl.semaphore_wait(barrier_sem, 2)

        xv = x_ref[:, :]
        scores = jnp.dot(xv, rw_ref[:, :], preferred_element_type=jnp.float32)
        m = jnp.max(scores, axis=1, keepdims=True)
        p = jnp.exp(scores - m)
        p = p / jnp.sum(p, axis=1, keepdims=True)
        e0 = idx_ref[:, 0]
        e1 = idx_ref[:, 1]
        ids = lax.broadcasted_iota(jnp.int32, (N_TOK, N_EXP), 1)
        g0 = jnp.sum(jnp.where(ids == e0[:, None], p, 0.0), axis=1)
        g1 = jnp.sum(jnp.where(ids == e1[:, None], p, 0.0), axis=1)
        gs = g0 + g1
        w0 = g0 / gs
        w1 = g1 / gs

        partial = jnp.zeros((N_TOK, H), jnp.float32)
        for j in range(e_local):
            ge = d * e_local + j
            wj = jnp.where(e0 == ge, w0, 0.0) + jnp.where(e1 == ge, w1, 0.0)
            xj = (xv * wj[:, None]).astype(jnp.bfloat16)
            partial = partial + jnp.dot(
                xj, ew_ref[j].astype(jnp.bfloat16),
                preferred_element_type=jnp.float32,
            )
        acc[...] = partial.reshape(N_DEV, CHUNK, H)

        for s in range(N_DEV - 1):
            c_send = lax.rem(d - s + N_DEV, N_DEV)
            rdma = pltpu.make_async_remote_copy(
                src_ref=acc.at[c_send],
                dst_ref=rs_buf.at[s],
                send_sem=rs_send.at[s],
                recv_sem=rs_recv.at[s],
                device_id=(right,),
                device_id_type=pl.DeviceIdType.MESH,
            )
            rdma.start()
            rdma.wait()
            c_recv = lax.rem(d - s - 1 + N_DEV, N_DEV)
            acc[c_recv] = acc[c_recv] + rs_buf[s]

        own = lax.rem(d + 1, N_DEV)
        out_ref[pl.ds(own * CHUNK, CHUNK), :] = acc[own]

        for h in range(N_DEV - 1):
            c = lax.rem(d + 1 - h + N_DEV, N_DEV)
            rdma = pltpu.make_async_remote_copy(
                src_ref=out_ref.at[pl.ds(c * CHUNK, CHUNK), :],
                dst_ref=out_ref.at[pl.ds(c * CHUNK, CHUNK), :],
                send_sem=ag_send.at[h],
                recv_sem=ag_recv.at[h],
                device_id=(right,),
                device_id_type=pl.DeviceIdType.MESH,
            )
            rdma.start()
            rdma.wait()

    return pl.pallas_call(
        body,
        out_shape=jax.ShapeDtypeStruct((N_TOK, H), jnp.float32),
        in_specs=[pl.BlockSpec(memory_space=pltpu.VMEM)] * 4,
        out_specs=pl.BlockSpec(memory_space=pltpu.VMEM),
        scratch_shapes=[
            pltpu.VMEM((N_DEV, CHUNK, H), jnp.float32),
            pltpu.VMEM((N_DEV - 1, CHUNK, H), jnp.float32),
            pltpu.SemaphoreType.DMA((N_DEV - 1,)),
            pltpu.SemaphoreType.DMA((N_DEV - 1,)),
            pltpu.SemaphoreType.DMA((N_DEV - 1,)),
            pltpu.SemaphoreType.DMA((N_DEV - 1,)),
        ],
        compiler_params=pltpu.CompilerParams(collective_id=0),
    )(x, router_W, route_idx, expert_W)


# device time: 49797 ns/iter; 2.2041x vs baseline; 2.2041x over previous
import jax
import jax.numpy as jnp
from jax import lax
from jax.experimental import pallas as pl
from jax.experimental.pallas import tpu as pltpu

N_DEV = 16
N_TOK = 1024
D_MODEL = 256
N_EXP = 64
H = 512
CHUNK = N_TOK // N_DEV

MASKS = [1, 2, 4, 8]


def _rs_send_lists():
    lists = []
    for s, m in enumerate(MASKS):
        future = MASKS[s + 1:]
        rs = []
        for bits in range(1 << len(future)):
            r = m
            for j, fm in enumerate(future):
                if bits >> j & 1:
                    r |= fm
            rs.append(r)
        lists.append(sorted(rs))
    return lists


def _ag_send_lists():
    owned = [0]
    lists = []
    for m in reversed(MASKS):
        lists.append(list(owned))
        owned = owned + [r ^ m for r in owned]
    return lists


RS_SENDS = _rs_send_lists()
AG_SENDS = _ag_send_lists()
N_HOPS = N_DEV - 1


def kernel(x, router_W, route_idx, expert_W):
    e_local = expert_W.shape[0]

    def body(x_ref, rw_ref, idx_ref, ew_ref, out_ref,
             acc, send_bf, rs_buf, allb,
             rs_send, rs_recv, ag_send, ag_recv):
        d = lax.axis_index("i")

        barrier_sem = pltpu.get_barrier_semaphore()
        for m in MASKS:
            pl.semaphore_signal(
                barrier_sem, inc=1,
                device_id=(d ^ m,), device_id_type=pl.DeviceIdType.MESH,
            )
        pl.semaphore_wait(barrier_sem, len(MASKS))

        xv = x_ref[:, :]
        scores = jnp.dot(xv, rw_ref[:, :], preferred_element_type=jnp.float32)
        mx = jnp.max(scores, axis=1, keepdims=True)
        p = jnp.exp(scores - mx)
        p = p / jnp.sum(p, axis=1, keepdims=True)
        e0 = idx_ref[:, 0]
        e1 = idx_ref[:, 1]
        ids = lax.broadcasted_iota(jnp.int32, (N_TOK, N_EXP), 1)
        g0 = jnp.sum(jnp.where(ids == e0[:, None], p, 0.0), axis=1)
        g1 = jnp.sum(jnp.where(ids == e1[:, None], p, 0.0), axis=1)
        gs = g0 + g1
        w0 = g0 / gs
        w1 = g1 / gs

        partial = jnp.zeros((N_TOK, H), jnp.float32)
        for j in range(e_local):
            ge = d * e_local + j
            wj = jnp.where(e0 == ge, w0, 0.0) + jnp.where(e1 == ge, w1, 0.0)
            xj = (xv * wj[:, None]).astype(jnp.bfloat16)
            partial = partial + jnp.dot(
                xj, ew_ref[j].astype(jnp.bfloat16),
                preferred_element_type=jnp.float32,
            )
        acc[...] = partial.reshape(N_DEV, CHUNK, H)

        slot = 0
        for s, m in enumerate(MASKS):
            partner = d ^ m
            rdmas = []
            base = slot
            for k, r in enumerate(RS_SENDS[s]):
                c = d ^ r
                send_bf[c] = acc[c].astype(jnp.bfloat16)
                rdma = pltpu.make_async_remote_copy(
                    src_ref=send_bf.at[c],
                    dst_ref=rs_buf.at[base + k],
                    send_sem=rs_send.at[base + k],
                    recv_sem=rs_recv.at[base + k],
                    device_id=(partner,),
                    device_id_type=pl.DeviceIdType.MESH,
                )
                rdma.start()
                rdmas.append(rdma)
                slot += 1
            for k, r in enumerate(RS_SENDS[s]):
                rdmas[k].wait()
                c = d ^ r ^ m
                acc[c] = acc[c] + rs_buf[base + k].astype(jnp.float32)

        allb[d] = acc[d].astype(jnp.bfloat16)

        slot = 0
        for s, m in enumerate(reversed(MASKS)):
            partner = d ^ m
            rdmas = []
            base = slot
            for k, r in enumerate(AG_SENDS[s]):
                c = d ^ r
                rdma = pltpu.make_async_remote_copy(
                    src_ref=allb.at[c],
                    dst_ref=allb.at[c],
                    send_sem=ag_send.at[base + k],
                    recv_sem=ag_recv.at[base + k],
                    device_id=(partner,),
                    device_id_type=pl.DeviceIdType.MESH,
                )
                rdma.start()
                rdmas.append(rdma)
                slot += 1
            for rdma in rdmas:
                rdma.wait()

        out_ref[...] = allb[...].astype(jnp.float32).reshape(N_TOK, H)
        out_ref[pl.ds(d * CHUNK, CHUNK), :] = acc[d]

    return pl.pallas_call(
        body,
        out_shape=jax.ShapeDtypeStruct((N_TOK, H), jnp.float32),
        in_specs=[pl.BlockSpec(memory_space=pltpu.VMEM)] * 4,
        out_specs=pl.BlockSpec(memory_space=pltpu.VMEM),
        scratch_shapes=[
            pltpu.VMEM((N_DEV, CHUNK, H), jnp.float32),
            pltpu.VMEM((N_DEV, CHUNK, H), jnp.bfloat16),
            pltpu.VMEM((N_HOPS, CHUNK, H), jnp.bfloat16),
            pltpu.VMEM((N_DEV, CHUNK, H), jnp.bfloat16),
            pltpu.SemaphoreType.DMA((N_HOPS,)),
            pltpu.SemaphoreType.DMA((N_HOPS,)),
            pltpu.SemaphoreType.DMA((N_HOPS,)),
            pltpu.SemaphoreType.DMA((N_HOPS,)),
        ],
        compiler_params=pltpu.CompilerParams(collective_id=0),
    )(x, router_W, route_idx, expert_W)


# device time: 37638 ns/iter; 2.9162x vs baseline; 1.3231x over previous
import jax
import jax.numpy as jnp
from jax import lax
from jax.experimental import pallas as pl
from jax.experimental.pallas import tpu as pltpu

N_DEV = 16
N_TOK = 1024
D_MODEL = 256
N_EXP = 64
H = 512
CHUNK = N_TOK // N_DEV
N_PEER = N_DEV - 1


def kernel(x, router_W, route_idx, expert_W):
    e_local = expert_W.shape[0]

    def body(x_ref, rw_ref, idx_ref, ew_ref, out_ref,
             acc, send_bf, rs_buf, allb,
             rs_send, rs_recv, ag_send, ag_recv):
        d = lax.axis_index("i")

        barrier_sem = pltpu.get_barrier_semaphore()
        for k in range(N_PEER):
            q = lax.rem(d + 1 + k, N_DEV)
            pl.semaphore_signal(
                barrier_sem, inc=1,
                device_id=(q,), device_id_type=pl.DeviceIdType.MESH,
            )
        pl.semaphore_wait(barrier_sem, N_PEER)

        xv = x_ref[:, :]
        scores = jnp.dot(xv, rw_ref[:, :], preferred_element_type=jnp.float32)
        mx = jnp.max(scores, axis=1, keepdims=True)
        p = jnp.exp(scores - mx)
        p = p / jnp.sum(p, axis=1, keepdims=True)
        e0 = idx_ref[:, 0]
        e1 = idx_ref[:, 1]
        ids = lax.broadcasted_iota(jnp.int32, (N_TOK, N_EXP), 1)
        g0 = jnp.sum(jnp.where(ids == e0[:, None], p, 0.0), axis=1)
        g1 = jnp.sum(jnp.where(ids == e1[:, None], p, 0.0), axis=1)
        gs = g0 + g1
        w0 = g0 / gs
        w1 = g1 / gs

        partial = jnp.zeros((N_TOK, H), jnp.float32)
        for j in range(e_local):
            ge = d * e_local + j
            wj = jnp.where(e0 == ge, w0, 0.0) + jnp.where(e1 == ge, w1, 0.0)
            xj = (xv * wj[:, None]).astype(jnp.bfloat16)
            partial = partial + jnp.dot(
                xj, ew_ref[j].astype(jnp.bfloat16),
                preferred_element_type=jnp.float32,
            )
        acc[...] = partial.reshape(N_DEV, CHUNK, H)

        rs_rdmas = []
        for k in range(N_PEER):
            q = lax.rem(d + 1 + k, N_DEV)
            send_bf[q] = acc[q].astype(jnp.bfloat16)
            slot = N_PEER - 1 - k
            rdma = pltpu.make_async_remote_copy(
                src_ref=send_bf.at[q],
                dst_ref=rs_buf.at[slot],
                send_sem=rs_send.at[slot],
                recv_sem=rs_recv.at[slot],
                device_id=(q,),
                device_id_type=pl.DeviceIdType.MESH,
            )
            rdma.start()
            rs_rdmas.append(rdma)

        red = acc[d]
        for k in range(N_PEER):
            rs_rdmas[k].wait()
            slot = N_PEER - 1 - k
            red = red + rs_buf[slot].astype(jnp.float32)
        allb[d] = red.astype(jnp.bfloat16)

        ag_rdmas = []
        for k in range(N_PEER):
            q = lax.rem(d + 1 + k, N_DEV)
            slot = N_PEER - 1 - k
            rdma = pltpu.make_async_remote_copy(
                src_ref=allb.at[d],
                dst_ref=allb.at[d],
                send_sem=ag_send.at[slot],
                recv_sem=ag_recv.at[slot],
                device_id=(q,),
                device_id_type=pl.DeviceIdType.MESH,
            )
            rdma.start()
            ag_rdmas.append(rdma)
        for rdma in ag_rdmas:
            rdma.wait()

        out_ref[...] = allb[...].astype(jnp.float32).reshape(N_TOK, H)
        out_ref[pl.ds(d * CHUNK, CHUNK), :] = red

    return pl.pallas_call(
        body,
        out_shape=jax.ShapeDtypeStruct((N_TOK, H), jnp.float32),
        in_specs=[pl.BlockSpec(memory_space=pltpu.VMEM)] * 4,
        out_specs=pl.BlockSpec(memory_space=pltpu.VMEM),
        scratch_shapes=[
            pltpu.VMEM((N_DEV, CHUNK, H), jnp.float32),
            pltpu.VMEM((N_DEV, CHUNK, H), jnp.bfloat16),
            pltpu.VMEM((N_PEER, CHUNK, H), jnp.bfloat16),
            pltpu.VMEM((N_DEV, CHUNK, H), jnp.bfloat16),
            pltpu.SemaphoreType.DMA((N_PEER,)),
            pltpu.SemaphoreType.DMA((N_PEER,)),
            pltpu.SemaphoreType.DMA((N_PEER,)),
            pltpu.SemaphoreType.DMA((N_PEER,)),
        ],
        compiler_params=pltpu.CompilerParams(collective_id=0),
    )(x, router_W, route_idx, expert_W)


# device time: 34002 ns/iter; 3.2280x vs baseline; 1.1069x over previous
import jax
import jax.numpy as jnp
from jax import lax
from jax.experimental import pallas as pl
from jax.experimental.pallas import tpu as pltpu

N_DEV = 16
N_TOK = 1024
D_MODEL = 256
N_EXP = 64
H = 512
CHUNK = N_TOK // N_DEV
N_PEER = N_DEV - 1
N_GRP = 4
GRP_ROWS = N_TOK // N_GRP
GRP_CHUNKS = N_DEV // N_GRP


def kernel(x, router_W, route_idx, expert_W):
    e_local = expert_W.shape[0]

    def body(x_ref, rw_ref, idx_ref, ew_ref, out_ref,
             acc, send_bf, rs_buf, allb,
             rs_send, rs_recv, ag_send, ag_recv):
        d = lax.axis_index("i")
        my_plane = lax.div(d, GRP_CHUNKS)

        barrier_sem = pltpu.get_barrier_semaphore()
        for k in range(N_PEER):
            q = lax.rem(d + 1 + k, N_DEV)
            pl.semaphore_signal(
                barrier_sem, inc=1,
                device_id=(q,), device_id_type=pl.DeviceIdType.MESH,
            )

        ew_bf = [ew_ref[j].astype(jnp.bfloat16) for j in range(e_local)]
        rw = rw_ref[:, :]
        ids = lax.broadcasted_iota(jnp.int32, (GRP_ROWS, N_EXP), 1)

        send_waits = []
        for t in range(N_GRP):
            g = lax.rem(my_plane + 1 + t, N_GRP)
            row0 = g * GRP_ROWS
            xg = x_ref[pl.ds(row0, GRP_ROWS), :]
            scores = jnp.dot(xg, rw, preferred_element_type=jnp.float32)
            mx = jnp.max(scores, axis=1, keepdims=True)
            p = jnp.exp(scores - mx)
            p = p / jnp.sum(p, axis=1, keepdims=True)
            e0g = idx_ref[pl.ds(row0, GRP_ROWS), 0]
            e1g = idx_ref[pl.ds(row0, GRP_ROWS), 1]
            g0 = jnp.sum(jnp.where(ids == e0g[:, None], p, 0.0), axis=1)
            g1 = jnp.sum(jnp.where(ids == e1g[:, None], p, 0.0), axis=1)
            gs = g0 + g1
            w0g = g0 / gs
            w1g = g1 / gs
            pg = jnp.zeros((GRP_ROWS, H), jnp.float32)
            for j in range(e_local):
                ge = d * e_local + j
                wj = (jnp.where(e0g == ge, w0g, 0.0)
                      + jnp.where(e1g == ge, w1g, 0.0))
                xj = (xg * wj[:, None]).astype(jnp.bfloat16)
                pg = pg + jnp.dot(
                    xj, ew_bf[j], preferred_element_type=jnp.float32,
                )
            c0 = g * GRP_CHUNKS
            acc[pl.ds(c0, GRP_CHUNKS)] = pg.reshape(GRP_CHUNKS, CHUNK, H)
            send_bf[pl.ds(c0, GRP_CHUNKS)] = (
                pg.astype(jnp.bfloat16).reshape(GRP_CHUNKS, CHUNK, H)
            )
            if t == 0:
                pl.semaphore_wait(barrier_sem, N_PEER)
            for c_off in range(GRP_CHUNKS):
                c = c0 + c_off
                rdma = pltpu.make_async_remote_copy(
                    src_ref=send_bf.at[c],
                    dst_ref=rs_buf.at[lax.rem(d - c - 1 + N_DEV, N_DEV)],
                    send_sem=rs_send.at[t * GRP_CHUNKS + c_off],
                    recv_sem=rs_recv.at[lax.rem(d - c - 1 + N_DEV, N_DEV)],
                    device_id=(c,),
                    device_id_type=pl.DeviceIdType.MESH,
                )
                if t < N_GRP - 1:
                    rdma.start()
                    send_waits.append((rdma, None))
                else:
                    cond = c != d

                    @pl.when(cond)
                    def _(rdma=rdma):
                        rdma.start()

                    send_waits.append((rdma, cond))

        red = acc[d]
        for k in range(N_PEER):
            recv = pltpu.make_async_remote_copy(
                src_ref=send_bf.at[0],
                dst_ref=rs_buf.at[k],
                send_sem=rs_send.at[0],
                recv_sem=rs_recv.at[k],
                device_id=(d,),
                device_id_type=pl.DeviceIdType.MESH,
            )
            recv.wait_recv()
            red = red + rs_buf[k].astype(jnp.float32)
        allb[d] = red.astype(jnp.bfloat16)

        ag_rdmas = []
        for k in range(N_PEER):
            q = lax.rem(d + 1 + k, N_DEV)
            slot = N_PEER - 1 - k
            rdma = pltpu.make_async_remote_copy(
                src_ref=allb.at[d],
                dst_ref=allb.at[d],
                send_sem=ag_send.at[slot],
                recv_sem=ag_recv.at[slot],
                device_id=(q,),
                device_id_type=pl.DeviceIdType.MESH,
            )
            rdma.start()
            ag_rdmas.append(rdma)

        for rdma, cond in send_waits:
            if cond is None:
                rdma.wait_send()
            else:
                @pl.when(cond)
                def _(rdma=rdma):
                    rdma.wait_send()

        for rdma in ag_rdmas:
            rdma.wait()

        out_ref[...] = allb[...].astype(jnp.float32).reshape(N_TOK, H)
        out_ref[pl.ds(d * CHUNK, CHUNK), :] = red

    return pl.pallas_call(
        body,
        out_shape=jax.ShapeDtypeStruct((N_TOK, H), jnp.float32),
        in_specs=[pl.BlockSpec(memory_space=pltpu.VMEM)] * 4,
        out_specs=pl.BlockSpec(memory_space=pltpu.VMEM),
        scratch_shapes=[
            pltpu.VMEM((N_DEV, CHUNK, H), jnp.float32),
            pltpu.VMEM((N_DEV, CHUNK, H), jnp.bfloat16),
            pltpu.VMEM((N_DEV, CHUNK, H), jnp.bfloat16),
            pltpu.VMEM((N_DEV, CHUNK, H), jnp.bfloat16),
            pltpu.SemaphoreType.DMA((N_DEV,)),
            pltpu.SemaphoreType.DMA((N_DEV,)),
            pltpu.SemaphoreType.DMA((N_PEER,)),
            pltpu.SemaphoreType.DMA((N_PEER,)),
        ],
        compiler_params=pltpu.CompilerParams(collective_id=0),
    )(x, router_W, route_idx, expert_W)
